# Initial kernel scaffold; baseline (speedup 1.0000x reference)
import jax
import jax.numpy as jnp
from jax.experimental import pallas as pl


def kernel(features, edge_index, W1, b1, W2, b2, g1, be1, g2, be2, Wl0, bl0, Wl1, bl1, Wl2, bl2):
    raise NotImplementedError("write your pallas kernel here")


# speedup vs baseline: 6.3006x; 6.3006x over previous
import functools

import jax
import jax.numpy as jnp
from jax import lax
from jax.experimental import pallas as pl
from jax.experimental.pallas import tpu as pltpu
from jax.experimental.pallas import tpu_sc as plsc

N = 10000
E = 320000
F = 128
H = 128
C = 64

NC = 2
NS = 16
NW = NC * NS
CB = 80
ECH = E // NW // CB
DCH = E // NS // CB
NCP = 10
CPT = N // NCP

BLK = 2000
NBLK = N // BLK
_EPS = 1e-5
_HIGH = jax.lax.Precision.HIGHEST


_MESH = plsc.VectorSubcoreMesh(core_axis_name="c", subcore_axis_name="s")


def _deg_body(src_hbm, dst_hbm, ones_hbm, zeros_hbm, out_hbm,
              idx_v, ones_v, hist_sh):
    c = lax.axis_index("c")
    s = lax.axis_index("s")

    @pl.when(s < NCP)
    def _():
        pltpu.sync_copy(zeros_hbm, hist_sh.at[pl.ds(s * CPT, CPT)])

    pltpu.sync_copy(ones_hbm, ones_v)

    @pl.when(c == 0)
    def _():
        pltpu.sync_copy(src_hbm.at[s], idx_v)

    @pl.when(c == 1)
    def _():
        pltpu.sync_copy(dst_hbm.at[s], idx_v)

    plsc.subcore_barrier()

    def body(j, carry):
        pltpu.sync_copy(ones_v, hist_sh.at[idx_v.at[j, 0]], add=True)
        return carry

    lax.fori_loop(0, DCH, body, 0)
    plsc.subcore_barrier()

    @pl.when(s < NCP)
    def _():
        pltpu.sync_copy(hist_sh.at[pl.ds(s * CPT, CPT)],
                        out_hbm.at[c].at[pl.ds(s * CPT, CPT)])


_deg_call = pl.kernel(
    _deg_body,
    out_type=jax.ShapeDtypeStruct((NC, N, H), jnp.float32),
    mesh=_MESH,
    scratch_types=[
        pltpu.VMEM((DCH, 1, CB), jnp.int32),
        pltpu.VMEM((CB, H), jnp.float32),
        pltpu.VMEM_SHARED((N, H), jnp.float32),
    ],
)


def _edge_body(xw_hbm, src_hbm, dst_hbm, zeros_hbm, out_hbm,
               sidx_v, didx_v, rows_v, acc_sh):
    c = lax.axis_index("c")
    s = lax.axis_index("s")
    wid = c * NS + s

    @pl.when(s < NCP)
    def _():
        pltpu.sync_copy(zeros_hbm, acc_sh.at[pl.ds(s * CPT, CPT)])

    pltpu.sync_copy(src_hbm.at[wid], sidx_v)
    pltpu.sync_copy(dst_hbm.at[wid], didx_v)
    plsc.subcore_barrier()

    def body(j, carry):
        pltpu.sync_copy(xw_hbm.at[sidx_v.at[j, 0]], rows_v)
        pltpu.sync_copy(rows_v, acc_sh.at[didx_v.at[j, 0]], add=True)
        return carry

    lax.fori_loop(0, ECH, body, 0)
    plsc.subcore_barrier()

    @pl.when(s < NCP)
    def _():
        pltpu.sync_copy(acc_sh.at[pl.ds(s * CPT, CPT)],
                        out_hbm.at[c].at[pl.ds(s * CPT, CPT)])


_edge_call = pl.kernel(
    _edge_body,
    out_type=jax.ShapeDtypeStruct((NC, N, H), jnp.float32),
    mesh=_MESH,
    scratch_types=[
        pltpu.VMEM((ECH, 1, CB), jnp.int32),
        pltpu.VMEM((ECH, 1, CB), jnp.int32),
        pltpu.VMEM((CB, H), jnp.float32),
        pltpu.VMEM_SHARED((N, H), jnp.float32),
    ],
)


def _norm_from_deg(deg):
    return jnp.where(deg > 0, lax.rsqrt(jnp.maximum(deg, 1.0)), 0.0)


def _mm1_body(x_ref, deg_ref, w_ref, out_ref, pool_ref):
    i = pl.program_id(0)
    x = x_ref[...]

    @pl.when(i == 0)
    def _():
        pool_ref[...] = jnp.zeros_like(pool_ref)

    pool_ref[...] += jnp.sum(x, axis=0, keepdims=True)
    norm = _norm_from_deg(deg_ref[0, :, 0:1])
    out_ref[...] = jnp.dot(x * norm, w_ref[...],
                           preferred_element_type=jnp.float32, precision=_HIGH)


_mm1_call = pl.pallas_call(
    _mm1_body,
    grid=(NBLK,),
    in_specs=[
        pl.BlockSpec((BLK, F), lambda i: (i, 0)),
        pl.BlockSpec((1, BLK, H), lambda i: (0, i, 0)),
        pl.BlockSpec((F, H), lambda i: (0, 0)),
    ],
    out_specs=[
        pl.BlockSpec((BLK, H), lambda i: (i, 0)),
        pl.BlockSpec((1, H), lambda i: (0, 0)),
    ],
    out_shape=[
        jax.ShapeDtypeStruct((N, H), jnp.float32),
        jax.ShapeDtypeStruct((1, H), jnp.float32),
    ],
)


def _stats_body(p_ref, deg_ref, b_ref, pre_ref, s1_ref, s2_ref):
    i = pl.program_id(0)
    ps = p_ref[0] + p_ref[1]
    norm = _norm_from_deg(deg_ref[0, :, 0:1])
    pre = ps * norm + b_ref[...]
    pre_ref[...] = pre

    @pl.when(i == 0)
    def _():
        s1_ref[...] = jnp.zeros_like(s1_ref)
        s2_ref[...] = jnp.zeros_like(s2_ref)

    s1_ref[...] += jnp.sum(pre, axis=0, keepdims=True)
    s2_ref[...] += jnp.sum(pre * pre, axis=0, keepdims=True)


_stats_call = pl.pallas_call(
    _stats_body,
    grid=(NBLK,),
    in_specs=[
        pl.BlockSpec((NC, BLK, H), lambda i: (0, i, 0)),
        pl.BlockSpec((1, BLK, H), lambda i: (1, i, 0)),
        pl.BlockSpec((1, H), lambda i: (0, 0)),
    ],
    out_specs=[
        pl.BlockSpec((BLK, H), lambda i: (i, 0)),
        pl.BlockSpec((1, H), lambda i: (0, 0)),
        pl.BlockSpec((1, H), lambda i: (0, 0)),
    ],
    out_shape=[
        jax.ShapeDtypeStruct((N, H), jnp.float32),
        jax.ShapeDtypeStruct((1, H), jnp.float32),
        jax.ShapeDtypeStruct((1, H), jnp.float32),
    ],
)


def _bn_relu(pre, s1, s2, g, be):
    mu = s1 * (1.0 / N)
    var = jnp.maximum(s2 * (1.0 / N) - mu * mu, 0.0)
    inv = lax.rsqrt(var + _EPS)
    return jnp.maximum((pre - mu) * inv * g + be, 0.0)


def _l2_body(pre_ref, s1_ref, s2_ref, g_ref, be_ref, deg_ref, w_ref,
             out_ref, pool_ref):
    i = pl.program_id(0)
    hb = _bn_relu(pre_ref[...], s1_ref[...], s2_ref[...], g_ref[...], be_ref[...])

    @pl.when(i == 0)
    def _():
        pool_ref[...] = jnp.zeros_like(pool_ref)

    pool_ref[...] += jnp.sum(hb, axis=0, keepdims=True)
    norm = _norm_from_deg(deg_ref[0, :, 0:1])
    out_ref[...] = jnp.dot(hb * norm, w_ref[...],
                           preferred_element_type=jnp.float32, precision=_HIGH)


_l2_call = pl.pallas_call(
    _l2_body,
    grid=(NBLK,),
    in_specs=[
        pl.BlockSpec((BLK, H), lambda i: (i, 0)),
        pl.BlockSpec((1, H), lambda i: (0, 0)),
        pl.BlockSpec((1, H), lambda i: (0, 0)),
        pl.BlockSpec((1, H), lambda i: (0, 0)),
        pl.BlockSpec((1, H), lambda i: (0, 0)),
        pl.BlockSpec((1, BLK, H), lambda i: (0, i, 0)),
        pl.BlockSpec((H, H), lambda i: (0, 0)),
    ],
    out_specs=[
        pl.BlockSpec((BLK, H), lambda i: (i, 0)),
        pl.BlockSpec((1, H), lambda i: (0, 0)),
    ],
    out_shape=[
        jax.ShapeDtypeStruct((N, H), jnp.float32),
        jax.ShapeDtypeStruct((1, H), jnp.float32),
    ],
)


def _fin_body(pre_ref, s1_ref, s2_ref, g_ref, be_ref, p0_ref, p1_ref,
              wl0_ref, wl1_ref, wl2_ref, bl0_ref, bl1_ref, bl2_ref,
              scores_ref, pools_ref, acc):
    i = pl.program_id(0)
    h2 = _bn_relu(pre_ref[...], s1_ref[...], s2_ref[...], g_ref[...], be_ref[...])

    @pl.when(i == 0)
    def _():
        acc[...] = jnp.zeros_like(acc)

    acc[...] += jnp.sum(h2, axis=0, keepdims=True)

    @pl.when(i == NBLK - 1)
    def _():
        pools = jnp.concatenate([p0_ref[...], p1_ref[...], acc[...]], axis=0)
        pools = pools * (1.0 / N)
        pools_ref[...] = pools
        scores = (jnp.dot(pools[0:1], wl0_ref[...],
                          preferred_element_type=jnp.float32, precision=_HIGH)
                  + jnp.dot(pools[1:2], wl1_ref[...],
                            preferred_element_type=jnp.float32, precision=_HIGH)
                  + jnp.dot(pools[2:3], wl2_ref[...],
                            preferred_element_type=jnp.float32, precision=_HIGH)
                  + bl0_ref[...] + bl1_ref[...] + bl2_ref[...])
        scores_ref[...] = scores


_fin_call = pl.pallas_call(
    _fin_body,
    grid=(NBLK,),
    in_specs=[
        pl.BlockSpec((BLK, H), lambda i: (i, 0)),
        pl.BlockSpec((1, H), lambda i: (0, 0)),
        pl.BlockSpec((1, H), lambda i: (0, 0)),
        pl.BlockSpec((1, H), lambda i: (0, 0)),
        pl.BlockSpec((1, H), lambda i: (0, 0)),
        pl.BlockSpec((1, H), lambda i: (0, 0)),
        pl.BlockSpec((1, H), lambda i: (0, 0)),
        pl.BlockSpec((F, C), lambda i: (0, 0)),
        pl.BlockSpec((H, C), lambda i: (0, 0)),
        pl.BlockSpec((H, C), lambda i: (0, 0)),
        pl.BlockSpec((1, C), lambda i: (0, 0)),
        pl.BlockSpec((1, C), lambda i: (0, 0)),
        pl.BlockSpec((1, C), lambda i: (0, 0)),
    ],
    out_specs=[
        pl.BlockSpec((1, C), lambda i: (0, 0)),
        pl.BlockSpec((3, H), lambda i: (0, 0)),
    ],
    out_shape=[
        jax.ShapeDtypeStruct((1, C), jnp.float32),
        jax.ShapeDtypeStruct((3, H), jnp.float32),
    ],
    scratch_shapes=[pltpu.VMEM((1, H), jnp.float32)],
)


def kernel(features, edge_index, W1, b1, W2, b2, g1, be1, g2, be2,
           Wl0, bl0, Wl1, bl1, Wl2, bl2):
    src = edge_index[0]
    dst = edge_index[1]
    src_d = src.reshape(NS, DCH, 1, CB)
    dst_d = dst.reshape(NS, DCH, 1, CB)
    src_e = src.reshape(NW, ECH, 1, CB)
    dst_e = dst.reshape(NW, ECH, 1, CB)

    onesH = jnp.ones((CB, H), jnp.float32)
    zerosH = jnp.zeros((CPT, H), jnp.float32)

    b1r = b1.reshape(1, H)
    b2r = b2.reshape(1, H)
    g1r = g1.reshape(1, H)
    be1r = be1.reshape(1, H)
    g2r = g2.reshape(1, H)
    be2r = be2.reshape(1, H)
    bl0r = bl0.reshape(1, C)
    bl1r = bl1.reshape(1, C)
    bl2r = bl2.reshape(1, C)

    deg = _deg_call(src_d, dst_d, onesH, zerosH)

    xw1, pool0 = _mm1_call(features, deg, W1)
    p1 = _edge_call(xw1, src_e, dst_e, zerosH)
    pre1, s1a, s1b = _stats_call(p1, deg, b1r)
    xw2, pool1 = _l2_call(pre1, s1a, s1b, g1r, be1r, deg, W2)
    p2 = _edge_call(xw2, src_e, dst_e, zerosH)
    pre2, s2a, s2b = _stats_call(p2, deg, b2r)
    scores, pools = _fin_call(pre2, s2a, s2b, g2r, be2r, pool0, pool1,
                              Wl0, Wl1, Wl2, bl0r, bl1r, bl2r)
    return scores, pools.reshape(3, 1, H)


# speedup vs baseline: 7.9141x; 1.2561x over previous
import functools

import jax
import jax.numpy as jnp
from jax import lax
from jax.experimental import pallas as pl
from jax.experimental.pallas import tpu as pltpu
from jax.experimental.pallas import tpu_sc as plsc

N = 10000
E = 320000
F = 128
H = 128
C = 64

NC = 2
NS = 16
NW = NC * NS
CB = 80
DCH = E // NS // CB
EB = 100
ECH = E // NW // EB
PHC = ECH // 2
NCP = 10
CPT = N // NCP

BLK = 2000
NBLK = N // BLK
_EPS = 1e-5
_HIGH = jax.lax.Precision.HIGHEST


_MESH = plsc.VectorSubcoreMesh(core_axis_name="c", subcore_axis_name="s")


def _deg_body(src_hbm, dst_hbm, ones_hbm, zeros_hbm, out_hbm,
              idx_v, ones_v, hist_sh):
    c = lax.axis_index("c")
    s = lax.axis_index("s")

    @pl.when(s < NCP)
    def _():
        pltpu.sync_copy(zeros_hbm, hist_sh.at[pl.ds(s * CPT, CPT)])

    pltpu.sync_copy(ones_hbm, ones_v)

    @pl.when(c == 0)
    def _():
        pltpu.sync_copy(src_hbm.at[s], idx_v)

    @pl.when(c == 1)
    def _():
        pltpu.sync_copy(dst_hbm.at[s], idx_v)

    plsc.subcore_barrier()

    def body(j, carry):
        pltpu.sync_copy(ones_v, hist_sh.at[idx_v.at[j, 0]], add=True)
        return carry

    lax.fori_loop(0, DCH, body, 0)
    plsc.subcore_barrier()

    @pl.when(s < NCP)
    def _():
        pltpu.sync_copy(hist_sh.at[pl.ds(s * CPT, CPT)],
                        out_hbm.at[c].at[pl.ds(s * CPT, CPT)])


_deg_call = pl.kernel(
    _deg_body,
    out_type=jax.ShapeDtypeStruct((NC, N, H), jnp.float32),
    mesh=_MESH,
    scratch_types=[
        pltpu.VMEM((DCH, 1, CB), jnp.int32),
        pltpu.VMEM((CB, H), jnp.float32),
        pltpu.VMEM_SHARED((N, H), jnp.float32),
    ],
)


def _edge_body(xw_hbm, src_hbm, dst_hbm, zeros_hbm, out_hbm,
               sidx_v, didx_v, rows_v, acc_sh, sem):
    c = lax.axis_index("c")
    s = lax.axis_index("s")
    wid = c * NS + s

    @pl.when(s < NCP)
    def _():
        pltpu.sync_copy(zeros_hbm, acc_sh.at[pl.ds(s * CPT, CPT)])

    plsc.subcore_barrier()

    for p in range(2):
        if p > 0:
            pltpu.make_async_copy(rows_v.at[1],
                                  acc_sh.at[didx_v.at[PHC - 1, 0]], sem).wait()
        pltpu.sync_copy(src_hbm.at[wid, pl.ds(p * PHC, PHC)], sidx_v)
        pltpu.sync_copy(dst_hbm.at[wid, pl.ds(p * PHC, PHC)], didx_v)

        def body(l, carry):
            b = lax.rem(l, 2)
            pltpu.sync_copy(xw_hbm.at[sidx_v.at[l, 0]], rows_v.at[b])

            @pl.when(l > 0)
            def _():
                pltpu.make_async_copy(rows_v.at[1 - b],
                                      acc_sh.at[didx_v.at[l - 1, 0]], sem).wait()

            pltpu.async_copy(rows_v.at[b], acc_sh.at[didx_v.at[l, 0]], sem,
                             add=True)
            return carry

        lax.fori_loop(0, PHC, body, 0)

    pltpu.make_async_copy(rows_v.at[1], acc_sh.at[didx_v.at[PHC - 1, 0]],
                          sem).wait()
    plsc.subcore_barrier()

    @pl.when(s < NCP)
    def _():
        pltpu.sync_copy(acc_sh.at[pl.ds(s * CPT, CPT)],
                        out_hbm.at[c].at[pl.ds(s * CPT, CPT)])


_edge_call = pl.kernel(
    _edge_body,
    out_type=jax.ShapeDtypeStruct((NC, N, H), jnp.float32),
    mesh=_MESH,
    scratch_types=[
        pltpu.VMEM((PHC, 1, EB), jnp.int32),
        pltpu.VMEM((PHC, 1, EB), jnp.int32),
        pltpu.VMEM((2, EB, H), jnp.float32),
        pltpu.VMEM_SHARED((N, H), jnp.float32),
        pltpu.SemaphoreType.DMA,
    ],
)


def _norm_from_deg(deg):
    return jnp.where(deg > 0, lax.rsqrt(jnp.maximum(deg, 1.0)), 0.0)


def _mm1_body(x_ref, deg_ref, w_ref, out_ref, pool_ref):
    i = pl.program_id(0)
    x = x_ref[...]

    @pl.when(i == 0)
    def _():
        pool_ref[...] = jnp.zeros_like(pool_ref)

    pool_ref[...] += jnp.sum(x, axis=0, keepdims=True)
    norm = _norm_from_deg(deg_ref[0, :, 0:1])
    out_ref[...] = jnp.dot(x * norm, w_ref[...],
                           preferred_element_type=jnp.float32, precision=_HIGH)


_mm1_call = pl.pallas_call(
    _mm1_body,
    grid=(NBLK,),
    in_specs=[
        pl.BlockSpec((BLK, F), lambda i: (i, 0)),
        pl.BlockSpec((1, BLK, H), lambda i: (0, i, 0)),
        pl.BlockSpec((F, H), lambda i: (0, 0)),
    ],
    out_specs=[
        pl.BlockSpec((BLK, H), lambda i: (i, 0)),
        pl.BlockSpec((1, H), lambda i: (0, 0)),
    ],
    out_shape=[
        jax.ShapeDtypeStruct((N, H), jnp.float32),
        jax.ShapeDtypeStruct((1, H), jnp.float32),
    ],
)


def _stats_body(p_ref, deg_ref, b_ref, pre_ref, s1_ref, s2_ref):
    i = pl.program_id(0)
    ps = p_ref[0] + p_ref[1]
    norm = _norm_from_deg(deg_ref[0, :, 0:1])
    pre = ps * norm + b_ref[...]
    pre_ref[...] = pre

    @pl.when(i == 0)
    def _():
        s1_ref[...] = jnp.zeros_like(s1_ref)
        s2_ref[...] = jnp.zeros_like(s2_ref)

    s1_ref[...] += jnp.sum(pre, axis=0, keepdims=True)
    s2_ref[...] += jnp.sum(pre * pre, axis=0, keepdims=True)


_stats_call = pl.pallas_call(
    _stats_body,
    grid=(NBLK,),
    in_specs=[
        pl.BlockSpec((NC, BLK, H), lambda i: (0, i, 0)),
        pl.BlockSpec((1, BLK, H), lambda i: (1, i, 0)),
        pl.BlockSpec((1, H), lambda i: (0, 0)),
    ],
    out_specs=[
        pl.BlockSpec((BLK, H), lambda i: (i, 0)),
        pl.BlockSpec((1, H), lambda i: (0, 0)),
        pl.BlockSpec((1, H), lambda i: (0, 0)),
    ],
    out_shape=[
        jax.ShapeDtypeStruct((N, H), jnp.float32),
        jax.ShapeDtypeStruct((1, H), jnp.float32),
        jax.ShapeDtypeStruct((1, H), jnp.float32),
    ],
)


def _bn_relu(pre, s1, s2, g, be):
    mu = s1 * (1.0 / N)
    var = jnp.maximum(s2 * (1.0 / N) - mu * mu, 0.0)
    inv = lax.rsqrt(var + _EPS)
    return jnp.maximum((pre - mu) * inv * g + be, 0.0)


def _l2_body(pre_ref, s1_ref, s2_ref, g_ref, be_ref, deg_ref, w_ref,
             out_ref, pool_ref):
    i = pl.program_id(0)
    hb = _bn_relu(pre_ref[...], s1_ref[...], s2_ref[...], g_ref[...], be_ref[...])

    @pl.when(i == 0)
    def _():
        pool_ref[...] = jnp.zeros_like(pool_ref)

    pool_ref[...] += jnp.sum(hb, axis=0, keepdims=True)
    norm = _norm_from_deg(deg_ref[0, :, 0:1])
    out_ref[...] = jnp.dot(hb * norm, w_ref[...],
                           preferred_element_type=jnp.float32, precision=_HIGH)


_l2_call = pl.pallas_call(
    _l2_body,
    grid=(NBLK,),
    in_specs=[
        pl.BlockSpec((BLK, H), lambda i: (i, 0)),
        pl.BlockSpec((1, H), lambda i: (0, 0)),
        pl.BlockSpec((1, H), lambda i: (0, 0)),
        pl.BlockSpec((1, H), lambda i: (0, 0)),
        pl.BlockSpec((1, H), lambda i: (0, 0)),
        pl.BlockSpec((1, BLK, H), lambda i: (0, i, 0)),
        pl.BlockSpec((H, H), lambda i: (0, 0)),
    ],
    out_specs=[
        pl.BlockSpec((BLK, H), lambda i: (i, 0)),
        pl.BlockSpec((1, H), lambda i: (0, 0)),
    ],
    out_shape=[
        jax.ShapeDtypeStruct((N, H), jnp.float32),
        jax.ShapeDtypeStruct((1, H), jnp.float32),
    ],
)


def _fin_body(pre_ref, s1_ref, s2_ref, g_ref, be_ref, p0_ref, p1_ref,
              wl0_ref, wl1_ref, wl2_ref, bl0_ref, bl1_ref, bl2_ref,
              scores_ref, pools_ref, acc):
    i = pl.program_id(0)
    h2 = _bn_relu(pre_ref[...], s1_ref[...], s2_ref[...], g_ref[...], be_ref[...])

    @pl.when(i == 0)
    def _():
        acc[...] = jnp.zeros_like(acc)

    acc[...] += jnp.sum(h2, axis=0, keepdims=True)

    @pl.when(i == NBLK - 1)
    def _():
        pools = jnp.concatenate([p0_ref[...], p1_ref[...], acc[...]], axis=0)
        pools = pools * (1.0 / N)
        pools_ref[...] = pools
        scores = (jnp.dot(pools[0:1], wl0_ref[...],
                          preferred_element_type=jnp.float32, precision=_HIGH)
                  + jnp.dot(pools[1:2], wl1_ref[...],
                            preferred_element_type=jnp.float32, precision=_HIGH)
                  + jnp.dot(pools[2:3], wl2_ref[...],
                            preferred_element_type=jnp.float32, precision=_HIGH)
                  + bl0_ref[...] + bl1_ref[...] + bl2_ref[...])
        scores_ref[...] = scores


_fin_call = pl.pallas_call(
    _fin_body,
    grid=(NBLK,),
    in_specs=[
        pl.BlockSpec((BLK, H), lambda i: (i, 0)),
        pl.BlockSpec((1, H), lambda i: (0, 0)),
        pl.BlockSpec((1, H), lambda i: (0, 0)),
        pl.BlockSpec((1, H), lambda i: (0, 0)),
        pl.BlockSpec((1, H), lambda i: (0, 0)),
        pl.BlockSpec((1, H), lambda i: (0, 0)),
        pl.BlockSpec((1, H), lambda i: (0, 0)),
        pl.BlockSpec((F, C), lambda i: (0, 0)),
        pl.BlockSpec((H, C), lambda i: (0, 0)),
        pl.BlockSpec((H, C), lambda i: (0, 0)),
        pl.BlockSpec((1, C), lambda i: (0, 0)),
        pl.BlockSpec((1, C), lambda i: (0, 0)),
        pl.BlockSpec((1, C), lambda i: (0, 0)),
    ],
    out_specs=[
        pl.BlockSpec((1, C), lambda i: (0, 0)),
        pl.BlockSpec((3, H), lambda i: (0, 0)),
    ],
    out_shape=[
        jax.ShapeDtypeStruct((1, C), jnp.float32),
        jax.ShapeDtypeStruct((3, H), jnp.float32),
    ],
    scratch_shapes=[pltpu.VMEM((1, H), jnp.float32)],
)


def kernel(features, edge_index, W1, b1, W2, b2, g1, be1, g2, be2,
           Wl0, bl0, Wl1, bl1, Wl2, bl2):
    src = edge_index[0]
    dst = edge_index[1]
    src_d = src.reshape(NS, DCH, 1, CB)
    dst_d = dst.reshape(NS, DCH, 1, CB)
    src_e = src.reshape(NW, ECH, 1, EB)
    dst_e = dst.reshape(NW, ECH, 1, EB)

    onesH = jnp.ones((CB, H), jnp.float32)
    zerosH = jnp.zeros((CPT, H), jnp.float32)

    b1r = b1.reshape(1, H)
    b2r = b2.reshape(1, H)
    g1r = g1.reshape(1, H)
    be1r = be1.reshape(1, H)
    g2r = g2.reshape(1, H)
    be2r = be2.reshape(1, H)
    bl0r = bl0.reshape(1, C)
    bl1r = bl1.reshape(1, C)
    bl2r = bl2.reshape(1, C)

    deg = _deg_call(src_d, dst_d, onesH, zerosH)

    xw1, pool0 = _mm1_call(features, deg, W1)
    p1 = _edge_call(xw1, src_e, dst_e, zerosH)
    pre1, s1a, s1b = _stats_call(p1, deg, b1r)
    xw2, pool1 = _l2_call(pre1, s1a, s1b, g1r, be1r, deg, W2)
    p2 = _edge_call(xw2, src_e, dst_e, zerosH)
    pre2, s2a, s2b = _stats_call(p2, deg, b2r)
    scores, pools = _fin_call(pre2, s2a, s2b, g2r, be2r, pool0, pool1,
                              Wl0, Wl1, Wl2, bl0r, bl1r, bl2r)
    return scores, pools.reshape(3, 1, H)


# speedup vs baseline: 9.5058x; 1.2011x over previous
import functools

import jax
import jax.numpy as jnp
from jax import lax
from jax.experimental import pallas as pl
from jax.experimental.pallas import tpu as pltpu
from jax.experimental.pallas import tpu_sc as plsc

N = 10000
E = 320000
F = 128
H = 128
C = 64

NC = 2
NS = 16
NW = NC * NS
CB = 80
DCH = E // NS // CB
EB = 100
ECH = E // NW // EB
PHC = ECH // 2
NCP = 10
CPT = N // NCP
HR = N * 8 // 128
NRED = HR // 125

BLK = 2000
NBLK = N // BLK
_EPS = 1e-5
_HIGH = jax.lax.Precision.HIGHEST


_MESH = plsc.VectorSubcoreMesh(core_axis_name="c", subcore_axis_name="s")


def _deg_body(src_hbm, dst_hbm, rid_hbm, zeros_hbm, out_hbm,
              idx_v, rid_v, hist_v, tot_sh):
    c = lax.axis_index("c")
    s = lax.axis_index("s")

    @pl.when(s == 0)
    def _():
        pltpu.sync_copy(zeros_hbm, tot_sh)

    @pl.when(c == 0)
    def _():
        pltpu.sync_copy(src_hbm.at[s], idx_v)

    @pl.when(c == 1)
    def _():
        pltpu.sync_copy(dst_hbm.at[s], idx_v)

    pltpu.sync_copy(rid_hbm, rid_v)
    pltpu.sync_copy(zeros_hbm, hist_v)

    lanes = lax.iota(jnp.int32, 16)
    kcol = lanes & 7
    m_lo = lanes < 8
    m_hi = lanes >= 8
    one = jnp.ones((16,), jnp.float32)

    def body(r, carry):
        for kk in range(5):
            ii = idx_v[r, pl.ds(kk * 16, 16)]
            cell = ii * 8 + kcol
            row = lax.shift_right_logical(cell, 7)
            col = cell & 127
            va = plsc.load_gather(hist_v, [row, col], mask=m_lo)
            plsc.store_scatter(hist_v, [row, col], va + one, mask=m_lo)
            vb = plsc.load_gather(hist_v, [row, col], mask=m_hi)
            plsc.store_scatter(hist_v, [row, col], vb + one, mask=m_hi)
        return carry

    lax.fori_loop(0, DCH, body, 0)
    plsc.subcore_barrier()

    def red(q, carry):
        pltpu.sync_copy(hist_v.at[pl.ds(q * 125, 125)],
                        tot_sh.at[rid_v.at[q, 0]], add=True)
        return carry

    lax.fori_loop(0, NRED, red, 0)
    plsc.subcore_barrier()

    @pl.when(s == 0)
    def _():
        pltpu.sync_copy(tot_sh, out_hbm.at[c])


_deg_call = pl.kernel(
    _deg_body,
    out_type=jax.ShapeDtypeStruct((NC, HR, 128), jnp.float32),
    mesh=_MESH,
    compiler_params=pltpu.CompilerParams(needs_layout_passes=False),
    scratch_types=[
        pltpu.VMEM((DCH, CB), jnp.int32),
        pltpu.VMEM((NRED, 1, 125), jnp.int32),
        pltpu.VMEM((HR, 128), jnp.float32),
        pltpu.VMEM_SHARED((HR, 128), jnp.float32),
    ],
)


def _edge_body(xw_hbm, src_hbm, dst_hbm, zeros_hbm, out_hbm,
               sidx_v, didx_v, rows_v, acc_sh, sem):
    c = lax.axis_index("c")
    s = lax.axis_index("s")
    wid = c * NS + s

    @pl.when(s < NCP)
    def _():
        pltpu.sync_copy(zeros_hbm, acc_sh.at[pl.ds(s * CPT, CPT)])

    plsc.subcore_barrier()

    for p in range(2):
        if p > 0:
            pltpu.make_async_copy(rows_v.at[1],
                                  acc_sh.at[didx_v.at[PHC - 1, 0]], sem).wait()
        pltpu.sync_copy(src_hbm.at[wid, pl.ds(p * PHC, PHC)], sidx_v)
        pltpu.sync_copy(dst_hbm.at[wid, pl.ds(p * PHC, PHC)], didx_v)

        def body(l, carry):
            b = lax.rem(l, 2)
            pltpu.sync_copy(xw_hbm.at[sidx_v.at[l, 0]], rows_v.at[b])

            @pl.when(l > 0)
            def _():
                pltpu.make_async_copy(rows_v.at[1 - b],
                                      acc_sh.at[didx_v.at[l - 1, 0]], sem).wait()

            pltpu.async_copy(rows_v.at[b], acc_sh.at[didx_v.at[l, 0]], sem,
                             add=True)
            return carry

        lax.fori_loop(0, PHC, body, 0)

    pltpu.make_async_copy(rows_v.at[1], acc_sh.at[didx_v.at[PHC - 1, 0]],
                          sem).wait()
    plsc.subcore_barrier()

    @pl.when(s < NCP)
    def _():
        pltpu.sync_copy(acc_sh.at[pl.ds(s * CPT, CPT)],
                        out_hbm.at[c].at[pl.ds(s * CPT, CPT)])


_edge_call = pl.kernel(
    _edge_body,
    out_type=jax.ShapeDtypeStruct((NC, N, H), jnp.float32),
    mesh=_MESH,
    scratch_types=[
        pltpu.VMEM((PHC, 1, EB), jnp.int32),
        pltpu.VMEM((PHC, 1, EB), jnp.int32),
        pltpu.VMEM((2, EB, H), jnp.float32),
        pltpu.VMEM_SHARED((N, H), jnp.float32),
        pltpu.SemaphoreType.DMA,
    ],
)


def _norm_from_deg(deg):
    return jnp.where(deg > 0, lax.rsqrt(jnp.maximum(deg, 1.0)), 0.0)


def _mm1_body(x_ref, deg_ref, w_ref, out_ref, pool_ref):
    i = pl.program_id(0)
    x = x_ref[...]

    @pl.when(i == 0)
    def _():
        pool_ref[...] = jnp.zeros_like(pool_ref)

    pool_ref[...] += jnp.sum(x, axis=0, keepdims=True)
    norm = _norm_from_deg(jnp.sum(deg_ref[0], axis=-1, keepdims=True))
    out_ref[...] = jnp.dot(x * norm, w_ref[...],
                           preferred_element_type=jnp.float32, precision=_HIGH)


_mm1_call = pl.pallas_call(
    _mm1_body,
    grid=(NBLK,),
    in_specs=[
        pl.BlockSpec((BLK, F), lambda i: (i, 0)),
        pl.BlockSpec((1, BLK, 8), lambda i: (0, i, 0)),
        pl.BlockSpec((F, H), lambda i: (0, 0)),
    ],
    out_specs=[
        pl.BlockSpec((BLK, H), lambda i: (i, 0)),
        pl.BlockSpec((1, H), lambda i: (0, 0)),
    ],
    out_shape=[
        jax.ShapeDtypeStruct((N, H), jnp.float32),
        jax.ShapeDtypeStruct((1, H), jnp.float32),
    ],
)


def _stats_body(p_ref, deg_ref, b_ref, pre_ref, s1_ref, s2_ref):
    i = pl.program_id(0)
    ps = p_ref[0] + p_ref[1]
    norm = _norm_from_deg(jnp.sum(deg_ref[0], axis=-1, keepdims=True))
    pre = ps * norm + b_ref[...]
    pre_ref[...] = pre

    @pl.when(i == 0)
    def _():
        s1_ref[...] = jnp.zeros_like(s1_ref)
        s2_ref[...] = jnp.zeros_like(s2_ref)

    s1_ref[...] += jnp.sum(pre, axis=0, keepdims=True)
    s2_ref[...] += jnp.sum(pre * pre, axis=0, keepdims=True)


_stats_call = pl.pallas_call(
    _stats_body,
    grid=(NBLK,),
    in_specs=[
        pl.BlockSpec((NC, BLK, H), lambda i: (0, i, 0)),
        pl.BlockSpec((1, BLK, 8), lambda i: (1, i, 0)),
        pl.BlockSpec((1, H), lambda i: (0, 0)),
    ],
    out_specs=[
        pl.BlockSpec((BLK, H), lambda i: (i, 0)),
        pl.BlockSpec((1, H), lambda i: (0, 0)),
        pl.BlockSpec((1, H), lambda i: (0, 0)),
    ],
    out_shape=[
        jax.ShapeDtypeStruct((N, H), jnp.float32),
        jax.ShapeDtypeStruct((1, H), jnp.float32),
        jax.ShapeDtypeStruct((1, H), jnp.float32),
    ],
)


def _bn_relu(pre, s1, s2, g, be):
    mu = s1 * (1.0 / N)
    var = jnp.maximum(s2 * (1.0 / N) - mu * mu, 0.0)
    inv = lax.rsqrt(var + _EPS)
    return jnp.maximum((pre - mu) * inv * g + be, 0.0)


def _l2_body(pre_ref, s1_ref, s2_ref, g_ref, be_ref, deg_ref, w_ref,
             out_ref, pool_ref):
    i = pl.program_id(0)
    hb = _bn_relu(pre_ref[...], s1_ref[...], s2_ref[...], g_ref[...], be_ref[...])

    @pl.when(i == 0)
    def _():
        pool_ref[...] = jnp.zeros_like(pool_ref)

    pool_ref[...] += jnp.sum(hb, axis=0, keepdims=True)
    norm = _norm_from_deg(jnp.sum(deg_ref[0], axis=-1, keepdims=True))
    out_ref[...] = jnp.dot(hb * norm, w_ref[...],
                           preferred_element_type=jnp.float32, precision=_HIGH)


_l2_call = pl.pallas_call(
    _l2_body,
    grid=(NBLK,),
    in_specs=[
        pl.BlockSpec((BLK, H), lambda i: (i, 0)),
        pl.BlockSpec((1, H), lambda i: (0, 0)),
        pl.BlockSpec((1, H), lambda i: (0, 0)),
        pl.BlockSpec((1, H), lambda i: (0, 0)),
        pl.BlockSpec((1, H), lambda i: (0, 0)),
        pl.BlockSpec((1, BLK, 8), lambda i: (0, i, 0)),
        pl.BlockSpec((H, H), lambda i: (0, 0)),
    ],
    out_specs=[
        pl.BlockSpec((BLK, H), lambda i: (i, 0)),
        pl.BlockSpec((1, H), lambda i: (0, 0)),
    ],
    out_shape=[
        jax.ShapeDtypeStruct((N, H), jnp.float32),
        jax.ShapeDtypeStruct((1, H), jnp.float32),
    ],
)


def _fin_body(pre_ref, s1_ref, s2_ref, g_ref, be_ref, p0_ref, p1_ref,
              wl0_ref, wl1_ref, wl2_ref, bl0_ref, bl1_ref, bl2_ref,
              scores_ref, pools_ref, acc):
    i = pl.program_id(0)
    h2 = _bn_relu(pre_ref[...], s1_ref[...], s2_ref[...], g_ref[...], be_ref[...])

    @pl.when(i == 0)
    def _():
        acc[...] = jnp.zeros_like(acc)

    acc[...] += jnp.sum(h2, axis=0, keepdims=True)

    @pl.when(i == NBLK - 1)
    def _():
        pools = jnp.concatenate([p0_ref[...], p1_ref[...], acc[...]], axis=0)
        pools = pools * (1.0 / N)
        pools_ref[...] = pools
        scores = (jnp.dot(pools[0:1], wl0_ref[...],
                          preferred_element_type=jnp.float32, precision=_HIGH)
                  + jnp.dot(pools[1:2], wl1_ref[...],
                            preferred_element_type=jnp.float32, precision=_HIGH)
                  + jnp.dot(pools[2:3], wl2_ref[...],
                            preferred_element_type=jnp.float32, precision=_HIGH)
                  + bl0_ref[...] + bl1_ref[...] + bl2_ref[...])
        scores_ref[...] = scores


_fin_call = pl.pallas_call(
    _fin_body,
    grid=(NBLK,),
    in_specs=[
        pl.BlockSpec((BLK, H), lambda i: (i, 0)),
        pl.BlockSpec((1, H), lambda i: (0, 0)),
        pl.BlockSpec((1, H), lambda i: (0, 0)),
        pl.BlockSpec((1, H), lambda i: (0, 0)),
        pl.BlockSpec((1, H), lambda i: (0, 0)),
        pl.BlockSpec((1, H), lambda i: (0, 0)),
        pl.BlockSpec((1, H), lambda i: (0, 0)),
        pl.BlockSpec((F, C), lambda i: (0, 0)),
        pl.BlockSpec((H, C), lambda i: (0, 0)),
        pl.BlockSpec((H, C), lambda i: (0, 0)),
        pl.BlockSpec((1, C), lambda i: (0, 0)),
        pl.BlockSpec((1, C), lambda i: (0, 0)),
        pl.BlockSpec((1, C), lambda i: (0, 0)),
    ],
    out_specs=[
        pl.BlockSpec((1, C), lambda i: (0, 0)),
        pl.BlockSpec((3, H), lambda i: (0, 0)),
    ],
    out_shape=[
        jax.ShapeDtypeStruct((1, C), jnp.float32),
        jax.ShapeDtypeStruct((3, H), jnp.float32),
    ],
    scratch_shapes=[pltpu.VMEM((1, H), jnp.float32)],
)


def kernel(features, edge_index, W1, b1, W2, b2, g1, be1, g2, be2,
           Wl0, bl0, Wl1, bl1, Wl2, bl2):
    src = edge_index[0]
    dst = edge_index[1]
    src_d = src.reshape(NS, DCH, CB)
    dst_d = dst.reshape(NS, DCH, CB)
    src_e = src.reshape(NW, ECH, 1, EB)
    dst_e = dst.reshape(NW, ECH, 1, EB)

    rid = jnp.arange(HR, dtype=jnp.int32).reshape(NRED, 1, 125)
    zerosHR = jnp.zeros((HR, 128), jnp.float32)
    zerosH = jnp.zeros((CPT, H), jnp.float32)

    b1r = b1.reshape(1, H)
    b2r = b2.reshape(1, H)
    g1r = g1.reshape(1, H)
    be1r = be1.reshape(1, H)
    g2r = g2.reshape(1, H)
    be2r = be2.reshape(1, H)
    bl0r = bl0.reshape(1, C)
    bl1r = bl1.reshape(1, C)
    bl2r = bl2.reshape(1, C)

    deg = _deg_call(src_d, dst_d, rid, zerosHR)
    deg = deg.reshape(NC, N, 8)

    xw1, pool0 = _mm1_call(features, deg, W1)
    p1 = _edge_call(xw1, src_e, dst_e, zerosH)
    pre1, s1a, s1b = _stats_call(p1, deg, b1r)
    xw2, pool1 = _l2_call(pre1, s1a, s1b, g1r, be1r, deg, W2)
    p2 = _edge_call(xw2, src_e, dst_e, zerosH)
    pre2, s2a, s2b = _stats_call(p2, deg, b2r)
    scores, pools = _fin_call(pre2, s2a, s2b, g2r, be2r, pool0, pool1,
                              Wl0, Wl1, Wl2, bl0r, bl1r, bl2r)
    return scores, pools.reshape(3, 1, H)
